# baseline (speedup 1.0000x reference)
import functools

import jax
import jax.numpy as jnp
from jax import lax
from jax.experimental import pallas as pl
from jax.experimental.pallas import tpu as pltpu
from jax.experimental.pallas import tpu_sc as plsc

LANES = 16
CHUNK = 80


_VSTEP = 2048


def _transpose_body(x_ref, y_hbm, s0, s1, sem0, sem1):
    t = pl.program_id(0)
    v = x_ref.shape[2]
    nfull = v // _VSTEP
    tail = v - nfull * _VSTEP
    scr = (s0, s1)
    sems = (sem0, sem1)
    spans = [(j * _VSTEP, _VSTEP) for j in range(nfull)]
    if tail:
        spans.append((nfull * _VSTEP, tail))
    for j, (v0, w) in enumerate(spans):
        b = j % 2
        if j >= 2:
            pv0, pw = spans[j - 2]
            pltpu.make_async_copy(
                scr[b].at[pl.ds(0, pw)], y_hbm.at[t, pl.ds(pv0, pw)], sems[b]
            ).wait()
        eye = jnp.eye(x_ref.shape[1], dtype=jnp.float32)
        scr[b][pl.ds(0, w), :] = jax.lax.dot_general(
            x_ref[0, :, pl.ds(v0, w)],
            eye,
            dimension_numbers=(((0,), (0,)), ((), ())),
            precision=jax.lax.Precision.HIGHEST,
        )
        pltpu.make_async_copy(
            scr[b].at[pl.ds(0, w)], y_hbm.at[t, pl.ds(v0, w)], sems[b]
        ).start()
    for j in range(max(len(spans) - 2, 0), len(spans)):
        b = j % 2
        v0, w = spans[j]
        pltpu.make_async_copy(
            scr[b].at[pl.ds(0, w)], y_hbm.at[t, pl.ds(v0, w)], sems[b]
        ).wait()


def _tc_detile(tt):
    T, D, V = tt.shape
    return pl.pallas_call(
        _transpose_body,
        grid=(T,),
        in_specs=[pl.BlockSpec((1, D, V), lambda t: (t, 0, 0))],
        out_specs=pl.BlockSpec(memory_space=pl.ANY),
        out_shape=jax.ShapeDtypeStruct((T, V, D), jnp.float32),
        scratch_shapes=[
            pltpu.VMEM((_VSTEP, D), jnp.float32),
            pltpu.VMEM((_VSTEP, D), jnp.float32),
            pltpu.SemaphoreType.DMA,
            pltpu.SemaphoreType.DMA,
        ],
    )(tt)


def kernel(indices, tables):
    T, B, G = indices.shape
    V, D = tables.shape[1], tables.shape[2]
    info = plsc.get_sparse_core_info()
    NC, NS = info.num_cores, info.num_subcores
    NW = NC * NS
    bpw = B // NW
    rpw = bpw * G
    nct = rpw // CHUNK
    nck = T * nct
    bpc = CHUNK // G
    dh = D // LANES
    grp = 128 // D

    gtab = _tc_detile(jnp.transpose(tables, (0, 2, 1))).reshape((T * V) // grp, grp * D)
    idx_w = (
        indices.reshape(T, NW, rpw)
        .transpose(1, 0, 2)
        .reshape(NW, nck, CHUNK)
    )

    mesh = plsc.VectorSubcoreMesh(core_axis_name="c", subcore_axis_name="s")

    @functools.partial(
        pl.kernel,
        mesh=mesh,
        out_type=jax.ShapeDtypeStruct((B, T * D), jnp.float32),
        scratch_types=[
            pltpu.VMEM((nck, CHUNK), jnp.int32),
            pltpu.VMEM((2, CHUNK), jnp.int32),
            pltpu.VMEM((2, CHUNK), jnp.int32),
            pltpu.VMEM((2, CHUNK, grp * D), jnp.float32),
            pltpu.VMEM((bpw, T * D), jnp.float32),
            pltpu.SemaphoreType.DMA,
            pltpu.SemaphoreType.DMA,
        ],
    )
    def ebag(
        idx_hbm, tab_hbm, out_hbm, idx_v, gidx_v, off_v, gath_v, out_v, sem0, sem1
    ):
        wid = lax.axis_index("s") * NC + lax.axis_index("c")
        sems = (sem0, sem1)
        pltpu.sync_copy(idx_hbm.at[wid], idx_v)

        def issue(c, buf):
            t = c >> 3
            base = t * V
            for k in range(CHUNK // LANES):
                sl = pl.ds(k * LANES, LANES)
                x = idx_v[c, sl] + base
                gidx_v[buf, sl] = x >> 2
                off_v[buf, sl] = (x & (grp - 1)) * D
            pltpu.make_async_copy(
                tab_hbm.at[gidx_v.at[buf]], gath_v.at[buf], sems[buf]
            ).start()

        def drain(buf):
            pltpu.make_async_copy(
                tab_hbm.at[gidx_v.at[buf]], gath_v.at[buf], sems[buf]
            ).wait()

        def pool(c, buf):
            t = c >> 3
            s0 = (c & (nct - 1)) * bpc
            accs = {}
            for k in range(CHUNK // LANES):
                offs = off_v[buf, pl.ds(k * LANES, LANES)]
                for l in range(LANES):
                    r = k * LANES + l
                    b = r // G
                    off = offs[l]
                    for h in range(dh):
                        x = gath_v[buf, r, pl.ds(off + h * LANES, LANES)]
                        key = (b, h)
                        accs[key] = x if key not in accs else accs[key] + x
            for b in range(bpc):
                for h in range(dh):
                    out_v[s0 + b, pl.ds(t * D + h * LANES, LANES)] = accs[(b, h)]

        issue(0, 0)

        def pair_body(i, carry):
            c0 = 2 * i
            c1 = c0 + 1
            issue(c1, 1)
            drain(0)
            pool(c0, 0)

            @pl.when(c0 + 2 < nck)
            def _():
                issue(c0 + 2, 0)

            drain(1)
            pool(c1, 1)
            return carry

        lax.fori_loop(0, nck // 2, pair_body, 0)
        pltpu.sync_copy(out_v, out_hbm.at[pl.ds(wid * bpw, bpw)])

    return ebag(idx_w, gtab)


# speedup vs baseline: 2.9491x; 2.9491x over previous
import functools

import jax
import jax.numpy as jnp
from jax import lax
from jax.experimental import pallas as pl
from jax.experimental.pallas import tpu as pltpu
from jax.experimental.pallas import tpu_sc as plsc

LANES = 16
IDX_CHUNK = 128


def kernel(indices, tables):
    T, B, G = indices.shape
    V, D = tables.shape[1], tables.shape[2]
    info = plsc.get_sparse_core_info()
    NC, NS = info.num_cores, info.num_subcores
    NW = NC * NS
    bpw = B // NW
    rpw = bpw * G
    nch = rpw // IDX_CHUNK
    dh = D // LANES

    flat_tables = tables.reshape(T * V, D)
    idx_w = (
        indices.reshape(T, NW, rpw)
        .transpose(1, 0, 2)
        .reshape(NW, T * nch, IDX_CHUNK)
    )

    mesh = plsc.VectorSubcoreMesh(core_axis_name="c", subcore_axis_name="s")

    @functools.partial(
        pl.kernel,
        mesh=mesh,
        compiler_params=pltpu.CompilerParams(use_tc_tiling_on_sc=False),
        out_type=jax.ShapeDtypeStruct((B, T * D), jnp.float32),
        scratch_types=[
            pltpu.VMEM((T * nch, IDX_CHUNK), jnp.int32),
            pltpu.VMEM((2 * rpw, D), jnp.float32),
            pltpu.VMEM((bpw, T * D), jnp.float32),
            pltpu.SemaphoreType.DMA,
            pltpu.SemaphoreType.DMA,
        ],
    )
    def ebag(idx_hbm, tab_hbm, out_hbm, idx_v, rows_v, out_v, sem0, sem1):
        wid = lax.axis_index("s") * NC + lax.axis_index("c")
        sems = (sem0, sem1)
        pltpu.sync_copy(idx_hbm.at[wid], idx_v)

        def add_off(t):
            off = t * V
            for g in range(nch):
                row = t * nch + g
                for c in range(IDX_CHUNK // LANES):
                    sl = pl.ds(c * LANES, LANES)
                    idx_v[row, sl] = idx_v[row, sl] + off

        def copies(t, buf):
            return [
                pltpu.make_async_copy(
                    tab_hbm.at[idx_v.at[t * nch + g]],
                    rows_v.at[pl.ds(buf * rpw + g * IDX_CHUNK, IDX_CHUNK)],
                    sems[buf],
                )
                for g in range(nch)
            ]

        def issue(t, buf):
            add_off(t)
            for cp in copies(t, buf):
                cp.start()

        def drain(t, buf):
            for cp in copies(t, buf):
                cp.wait()

        def pool(t, buf):
            def sample_body(s, c2):
                base = buf * rpw + s * G
                for h in range(dh):
                    sl = pl.ds(h * LANES, LANES)
                    acc = rows_v[base, sl]
                    for j in range(1, G):
                        acc = acc + rows_v[base + j, sl]
                    out_v[s, pl.ds(t * D + h * LANES, LANES)] = acc
                return c2

            lax.fori_loop(0, bpw, sample_body, 0)

        issue(0, 0)

        def pair_body(i, carry):
            t0 = 2 * i
            t1 = t0 + 1
            issue(t1, 1)
            drain(t0, 0)
            pool(t0, 0)

            @pl.when(t0 + 2 < T)
            def _():
                issue(t0 + 2, 0)

            drain(t1, 1)
            pool(t1, 1)
            return carry

        lax.fori_loop(0, T // 2, pair_body, 0)
        pltpu.sync_copy(out_v, out_hbm.at[pl.ds(wid * bpw, bpw)])

    return ebag(idx_w, flat_tables)
